# baseline (speedup 1.0000x reference)
import functools

import jax
import jax.numpy as jnp
from jax import lax
from jax.experimental import pallas as pl
from jax.experimental.pallas import tpu as pltpu
from jax.experimental.pallas import tpu_sc as plsc

EMBED = 64
K = 1024
BETA = 0.25

B = 64
T = 576
N_TOKENS = B * T
RB = 8
BLOCK = RB * T

G = 2
BG = B // G
NG = BG * T

_NUM_CORES = 2
_NUM_SUBCORES = 16
_NW = _NUM_CORES * _NUM_SUBCORES
_B_PER_W = NG // _NW
_CHUNK = 96
_CHUNKS = _B_PER_W // _CHUNK


def _vq_body(x_ref, c_ref, codes2d_ref, codes_ref, loss_ref,
             codes_v, sem):
    i = pl.program_id(0)
    x = x_ref[...].reshape(BLOCK, EMBED)
    c = c_ref[...]
    c2 = jnp.sum(c * c, axis=1)
    x2 = jnp.sum(x * x, axis=1, keepdims=True)
    xc = jax.lax.dot_general(x, -2.0 * c, (((1,), (1,)), ((), ())),
                             preferred_element_type=jnp.float32)
    scores = (x2 + xc) + c2[None, :]
    minv = jnp.min(scores, axis=1, keepdims=True)
    iota = jax.lax.broadcasted_iota(jnp.int32, scores.shape, 1)
    codes = jnp.min(jnp.where(scores == minv, iota, K), axis=1)
    codes_v[...] = codes
    codes2d_ref[...] = codes.reshape(RB, T)
    pltpu.async_copy(codes_v, codes_ref.at[pl.ds(i * BLOCK, BLOCK)],
                     sem).wait()

    @pl.when(i == 0)
    def _init():
        loss_ref[0, 0] = 0.0

    loss_ref[0, 0] += jnp.sum(minv)


def _argmin_stage(x3, codebook, group):
    return pl.pallas_call(
        _vq_body,
        grid=(BG // RB,),
        in_specs=[
            pl.BlockSpec((RB, T, EMBED),
                         lambda i, g=group: (g * (BG // RB) + i, 0, 0)),
            pl.BlockSpec((K, EMBED), lambda i: (0, 0)),
        ],
        out_specs=[
            pl.BlockSpec((RB, T), lambda i: (i, 0)),
            pl.BlockSpec(memory_space=pl.ANY),
            pl.BlockSpec(block_shape=(1, 1), index_map=lambda i: (0, 0),
                         memory_space=pltpu.SMEM),
        ],
        out_shape=[
            jax.ShapeDtypeStruct((BG, T), jnp.int32),
            jax.ShapeDtypeStruct((NG,), jnp.int32),
            jax.ShapeDtypeStruct((1, 1), jnp.float32),
        ],
        scratch_shapes=[
            pltpu.VMEM((BLOCK,), jnp.int32),
            pltpu.SemaphoreType.DMA,
        ],
    )(x3, codebook)


@functools.cache
def _gather_stage():
    @functools.partial(
        pl.kernel,
        mesh=plsc.VectorSubcoreMesh(core_axis_name="c", subcore_axis_name="s"),
        out_type=jax.ShapeDtypeStruct((NG, EMBED), jnp.float32),
        scratch_types=[
            pltpu.VMEM((_B_PER_W,), jnp.int32),
            pltpu.VMEM((_CHUNKS, _CHUNK, EMBED), jnp.float32),
            pltpu.SemaphoreType.DMA,
            pltpu.SemaphoreType.DMA,
        ],
        compiler_params=pltpu.CompilerParams(use_tc_tiling_on_sc=False),
    )
    def gather(table_hbm, idx_hbm, out_hbm, idx_v, rows_v, sem_g, sem_w):
        wid = lax.axis_index("s") * _NUM_CORES + lax.axis_index("c")
        base = wid * _B_PER_W
        pltpu.sync_copy(idx_hbm.at[pl.ds(base, _B_PER_W)], idx_v)
        gathers = [
            pltpu.async_copy(table_hbm.at[idx_v.at[pl.ds(j * _CHUNK, _CHUNK)]],
                             rows_v.at[j], sem_g)
            for j in range(_CHUNKS)
        ]
        writes = []
        for j in range(_CHUNKS):
            gathers[j].wait()
            writes.append(pltpu.async_copy(
                rows_v.at[j],
                out_hbm.at[pl.ds(base + j * _CHUNK, _CHUNK)], sem_w))
        for w in writes:
            w.wait()

    return gather


def kernel(inputs, codebook):
    codes2d_g = []
    loss_acc = 0.0
    q = jnp.empty((B, T, EMBED), jnp.float32)
    for g in range(G):
        codes2d, codes, lacc = _argmin_stage(inputs, codebook, g)
        qg = _gather_stage()(codebook, codes)
        q = lax.dynamic_update_slice(q, qg.reshape(BG, T, EMBED),
                                     (g * BG, 0, 0))
        codes2d_g.append(codes2d)
        loss_acc = loss_acc + lacc[0, 0]
    codes_out = jnp.concatenate(codes2d_g, axis=0)
    loss = loss_acc * (1.0 + BETA) / (N_TOKENS * EMBED)
    return q, codes_out, loss


# speedup vs baseline: 1.0279x; 1.0279x over previous
import functools

import jax
import jax.numpy as jnp
from jax import lax
from jax.experimental import pallas as pl
from jax.experimental.pallas import tpu as pltpu
from jax.experimental.pallas import tpu_sc as plsc

EMBED = 64
K = 1024
BETA = 0.25

B = 64
T = 576
N_TOKENS = B * T
RB = 8
BLOCK = RB * T

_NUM_CORES = 2
_NUM_SUBCORES = 16
_NW = _NUM_CORES * _NUM_SUBCORES
_B_PER_W = N_TOKENS // _NW
_RB_PER_W = _B_PER_W // T
_CHUNK = 96
_CPR = T // _CHUNK
_CHUNKS = _B_PER_W // _CHUNK


def _vq_body(x_ref, c_ref, codes2d_ref, codes_ref, loss_ref,
             codes_v, sem):
    i = pl.program_id(0)
    x = x_ref[...].reshape(BLOCK, EMBED)
    c = c_ref[...]
    c2 = jnp.sum(c * c, axis=1)
    x2 = jnp.sum(x * x, axis=1, keepdims=True)
    xc = jax.lax.dot_general(x, -2.0 * c, (((1,), (1,)), ((), ())),
                             preferred_element_type=jnp.float32)
    scores = (x2 + xc) + c2[None, :]
    minv = jnp.min(scores, axis=1, keepdims=True)
    iota = jax.lax.broadcasted_iota(jnp.int32, scores.shape, 1)
    codes = jnp.min(jnp.where(scores == minv, iota, K), axis=1)
    codes_v[...] = codes
    codes2d_ref[...] = codes.reshape(RB, T)
    pltpu.async_copy(codes_v, codes_ref.at[pl.ds(i * BLOCK, BLOCK)],
                     sem).wait()

    @pl.when(i == 0)
    def _init():
        loss_ref[0, 0] = 0.0

    loss_ref[0, 0] += jnp.sum(minv)


def _argmin_stage(x3, codebook):
    return pl.pallas_call(
        _vq_body,
        grid=(B // RB,),
        in_specs=[
            pl.BlockSpec((RB, T, EMBED), lambda i: (i, 0, 0)),
            pl.BlockSpec((K, EMBED), lambda i: (0, 0)),
        ],
        out_specs=[
            pl.BlockSpec((RB, T), lambda i: (i, 0)),
            pl.BlockSpec(memory_space=pl.ANY),
            pl.BlockSpec(block_shape=(1, 1), index_map=lambda i: (0, 0),
                         memory_space=pltpu.SMEM),
        ],
        out_shape=[
            jax.ShapeDtypeStruct((B, T), jnp.int32),
            jax.ShapeDtypeStruct((N_TOKENS,), jnp.int32),
            jax.ShapeDtypeStruct((1, 1), jnp.float32),
        ],
        scratch_shapes=[
            pltpu.VMEM((BLOCK,), jnp.int32),
            pltpu.SemaphoreType.DMA,
        ],
    )(x3, codebook)


@functools.cache
def _gather_stage():
    @functools.partial(
        pl.kernel,
        mesh=plsc.VectorSubcoreMesh(core_axis_name="c", subcore_axis_name="s"),
        out_type=jax.ShapeDtypeStruct((B, T, EMBED), jnp.float32),
        scratch_types=[
            pltpu.VMEM((_B_PER_W,), jnp.int32),
            pltpu.VMEM((_CHUNKS, _CHUNK, EMBED), jnp.float32),
            pltpu.SemaphoreType.DMA,
            pltpu.SemaphoreType.DMA,
        ],
        compiler_params=pltpu.CompilerParams(use_tc_tiling_on_sc=False),
    )
    def gather(table_hbm, idx_hbm, out_hbm, idx_v, rows_v, sem_g, sem_w):
        wid = lax.axis_index("s") * _NUM_CORES + lax.axis_index("c")
        base = wid * _B_PER_W
        pltpu.sync_copy(idx_hbm.at[pl.ds(base, _B_PER_W)], idx_v)
        gathers = [
            pltpu.async_copy(table_hbm.at[idx_v.at[pl.ds(j * _CHUNK, _CHUNK)]],
                             rows_v.at[j], sem_g)
            for j in range(_CHUNKS)
        ]
        writes = []
        for j in range(_CHUNKS):
            r, k = divmod(j, _CPR)
            gathers[j].wait()
            writes.append(pltpu.async_copy(
                rows_v.at[j],
                out_hbm.at[_RB_PER_W * wid + r, pl.ds(k * _CHUNK, _CHUNK)],
                sem_w))
        for w in writes:
            w.wait()

    return gather


def kernel(inputs, codebook):
    codes2d, codes, loss_acc = _argmin_stage(inputs, codebook)
    q = _gather_stage()(codebook, codes)
    loss = loss_acc[0, 0] * (1.0 + BETA) / (N_TOKENS * EMBED)
    return q, codes2d, loss
